# baseline (speedup 1.0000x reference)
import functools

import jax
import jax.numpy as jnp
from jax import lax
from jax.experimental import pallas as pl
from jax.experimental.pallas import tpu as pltpu
from jax.experimental.pallas import tpu_sc as plsc

_MAXP = 2048
_H = 16
_S = 2048
_TBL = 2 * _MAXP - 1
_NW = 32
_ROWS_PER_W = (_H * _S) // _NW
_GROUPS_PER_W = _ROWS_PER_W // 8


def _rpb_body(src_hbm, out_hbm, sem):
    cid = lax.axis_index("c")
    sid = lax.axis_index("s")
    wid = sid * 2 + cid
    h = wid // 2
    half = wid % 2

    g0 = half * _GROUPS_PER_W

    def fire(g):
        u = 255 - g
        v = lax.rem(u, 16)
        m = lax.div(u, 16)
        pltpu.async_copy(
            src_hbm.at[h, v, pl.ds(m, 16), :, :],
            out_hbm.at[h, g],
            sem,
        )

    for p in range(7):
        fire(g0 + p)

    def step(k, carry):
        @pl.when(k < _GROUPS_PER_W - 7)
        def _():
            fire(g0 + k + 7)
        pltpu.make_async_copy(
            src_hbm.at[h, 0, pl.ds(0, 16), :, :], out_hbm.at[h, 0], sem
        ).wait()
        return carry

    lax.fori_loop(0, _GROUPS_PER_W, step, 0)


@jax.jit
def _rpb_sc(src):
    mesh = plsc.VectorSubcoreMesh(core_axis_name="c", subcore_axis_name="s")
    return pl.kernel(
        _rpb_body,
        out_type=jax.ShapeDtypeStruct((_H, _S // 8, _S // 128, 8, 128),
                                      jnp.float32),
        mesh=mesh,
        scratch_types=[
            pltpu.SemaphoreType.DMA,
        ],
        compiler_params=pltpu.CompilerParams(use_tc_tiling_on_sc=False),
    )(src)


def kernel(rel_pos_bias, seq_len):
    del seq_len
    cols = rel_pos_bias.T
    s1 = jnp.stack(
        [jnp.stack([cols[:, 8 * v + 7 - r: 8 * v + 7 - r + 3968]
                    for r in range(8)], axis=0)
         for v in range(16)], axis=0)
    src = s1.reshape(16, 8, 16, 31, 128).transpose(2, 0, 3, 1, 4)
    out5 = _rpb_sc(src)
    return out5.transpose(0, 1, 3, 2, 4).reshape(_H, _S, _S)


# speedup vs baseline: 1.0425x; 1.0425x over previous
import functools

import jax
import jax.numpy as jnp
from jax import lax
from jax.experimental import pallas as pl
from jax.experimental.pallas import tpu as pltpu
from jax.experimental.pallas import tpu_sc as plsc

_MAXP = 2048
_H = 16
_S = 2048
_TBL = 2 * _MAXP - 1
_NW = 32
_ROWS_PER_W = (_H * _S) // _NW
_GROUPS_PER_W = _ROWS_PER_W // 8


def _rpb_body(src_hbm, out_hbm, sem):
    cid = lax.axis_index("c")
    sid = lax.axis_index("s")
    wid = sid * 2 + cid
    h = wid // 2
    half = wid % 2

    g0 = half * _GROUPS_PER_W

    def fire(g):
        u = 255 - g
        v = lax.rem(u, 16)
        m = lax.div(u, 16)
        pltpu.async_copy(
            src_hbm.at[h, v, pl.ds(m, 16), :, :],
            out_hbm.at[h, g],
            sem,
        )

    for p in range(7):
        fire(g0 + p)

    def step(k, carry):
        @pl.when(k < _GROUPS_PER_W - 7)
        def _():
            fire(g0 + k + 7)
        pltpu.make_async_copy(
            src_hbm.at[h, 0, pl.ds(0, 16), :, :], out_hbm.at[h, 0], sem
        ).wait()
        return carry

    lax.fori_loop(0, _GROUPS_PER_W, step, 0)


@jax.jit
def _rpb_sc(src):
    mesh = plsc.VectorSubcoreMesh(core_axis_name="c", subcore_axis_name="s")
    return pl.kernel(
        _rpb_body,
        out_type=jax.ShapeDtypeStruct((_H, _S // 8, _S // 128, 8, 128),
                                      jnp.float32),
        mesh=mesh,
        scratch_types=[
            pltpu.SemaphoreType.DMA,
        ],
        compiler_params=pltpu.CompilerParams(use_tc_tiling_on_sc=False),
    )(src)


def kernel(rel_pos_bias, seq_len):
    del seq_len
    cols = rel_pos_bias.T
    colspad = jnp.pad(cols, ((0, 0), (0, 4103 - _TBL)))
    ws = jnp.stack([colspad[:, 7 - r:7 - r + 4096] for r in range(8)], axis=1)
    s2 = jnp.stack([ws[:, :, 8 * v: 8 * v + 3968] for v in range(16)], axis=1)
    src = s2.reshape(_H, 16, 8, 31, 128).transpose(0, 1, 3, 2, 4)
    out5 = _rpb_sc(src)
    return out5.transpose(0, 1, 3, 2, 4).reshape(_H, _S, _S)


# speedup vs baseline: 53.5709x; 51.3853x over previous
import functools

import jax
import jax.numpy as jnp
from jax import lax
from jax.experimental import pallas as pl
from jax.experimental.pallas import tpu as pltpu
from jax.experimental.pallas import tpu_sc as plsc

_MAXP = 2048
_H = 16
_S = 2048
_TBL = 2 * _MAXP - 1
_NW = 32
_ROWS_PER_W = (_H * _S) // _NW
_GROUPS_PER_W = _ROWS_PER_W // 8


def _rpb_body(src_hbm, out_hbm, st0, st1, sem):
    cid = lax.axis_index("c")
    sid = lax.axis_index("s")
    wid = sid * 2 + cid
    h = wid // 2
    half = wid % 2

    mlo = 8 * (1 - half)
    stages = (st0, st1)

    def drain_one_band():
        pltpu.make_async_copy(
            st0.at[pl.ds(0, 16), :, :], out_hbm.at[h, 0], sem
        ).wait()

    for v in range(16):
        buf = stages[v % 2]
        if v >= 2:
            for _q in range(8):
                drain_one_band()
        pltpu.sync_copy(src_hbm.at[h, v, pl.ds(mlo, 23), :, :], buf)
        for q in range(8):
            m = mlo + q
            g = 255 - (16 * m + v)
            pltpu.async_copy(
                buf.at[pl.ds(q, 16), :, :],
                out_hbm.at[h, g],
                sem,
            )
    for _q in range(16):
        drain_one_band()


@jax.jit
def _rpb_sc(src):
    mesh = plsc.VectorSubcoreMesh(core_axis_name="c", subcore_axis_name="s")
    return pl.kernel(
        _rpb_body,
        out_type=jax.ShapeDtypeStruct((_H, _S // 8, _S // 128, 8, 128),
                                      jnp.float32),
        mesh=mesh,
        scratch_types=[
            pltpu.VMEM((23, 8, 128), jnp.float32),
            pltpu.VMEM((23, 8, 128), jnp.float32),
            pltpu.SemaphoreType.DMA,
        ],
        compiler_params=pltpu.CompilerParams(use_tc_tiling_on_sc=False),
    )(src)


def kernel(rel_pos_bias, seq_len):
    del seq_len
    cols = rel_pos_bias.T
    colspad = jnp.pad(cols, ((0, 0), (0, 4103 - _TBL)))
    ws = jnp.stack([colspad[:, 7 - r:7 - r + 4096] for r in range(8)], axis=1)
    s2 = jnp.stack([ws[:, :, 8 * v: 8 * v + 3968] for v in range(16)], axis=1)
    src = s2.reshape(_H, 16, 8, 31, 128).transpose(0, 1, 3, 2, 4)
    out5 = _rpb_sc(src)
    return out5.transpose(0, 1, 3, 2, 4).reshape(_H, _S, _S)


# speedup vs baseline: 75.2710x; 1.4051x over previous
import functools

import jax
import jax.numpy as jnp
from jax import lax
from jax.experimental import pallas as pl
from jax.experimental.pallas import tpu as pltpu
from jax.experimental.pallas import tpu_sc as plsc

_MAXP = 2048
_H = 16
_S = 2048
_TBL = 2 * _MAXP - 1
_W = 4096
_GROUPS_PER_W = 128


def _rpb_body(ws_hbm, out_hbm, ws_v, sem):
    cid = lax.axis_index("c")
    sid = lax.axis_index("s")
    wid = sid * 2 + cid
    h = wid // 2
    half = wid % 2

    pltpu.sync_copy(ws_hbm.at[h], ws_v)

    g0 = half * _GROUPS_PER_W

    def fire(g):
        start = 8 * (255 - g)
        for J in range(16):
            pltpu.async_copy(
                ws_v.at[:, pl.ds(start + 128 * J, 128)],
                out_hbm.at[h, g, J],
                sem,
            )

    for p in range(3):
        fire(g0 + p)

    def step(k, carry):
        @pl.when(k < _GROUPS_PER_W - 3)
        def _():
            fire(g0 + k + 3)
        for _J in range(16):
            pltpu.make_async_copy(
                ws_v.at[:, pl.ds(0, 128)], out_hbm.at[h, 0, 0], sem
            ).wait()
        return carry

    lax.fori_loop(0, _GROUPS_PER_W, step, 0)


@jax.jit
def _rpb_sc(ws):
    mesh = plsc.VectorSubcoreMesh(core_axis_name="c", subcore_axis_name="s")
    return pl.kernel(
        _rpb_body,
        out_type=jax.ShapeDtypeStruct((_H, _S // 8, _S // 128, 8, 128),
                                      jnp.float32),
        mesh=mesh,
        scratch_types=[
            pltpu.VMEM((8, _W), jnp.float32),
            pltpu.SemaphoreType.DMA,
        ],
        compiler_params=pltpu.CompilerParams(use_tc_tiling_on_sc=False),
    )(ws)


def kernel(rel_pos_bias, seq_len):
    del seq_len
    cols = rel_pos_bias.T
    colspad = jnp.pad(cols, ((0, 0), (0, _W + 7 - _TBL)))
    ws = jnp.stack([colspad[:, 7 - r:7 - r + _W] for r in range(8)], axis=1)
    out5 = _rpb_sc(ws)
    return out5.transpose(0, 1, 3, 2, 4).reshape(_H, _S, _S)
